# baseline (speedup 1.0000x reference)
import jax
import jax.numpy as jnp
from jax import lax
from jax.experimental import pallas as pl
from jax.experimental.pallas import tpu as pltpu
from jax.experimental.pallas import tpu_sc as plsc

_NC = 2
_NS = 16
_NW = _NC * _NS
_B = 4096
_BPW = _B // _NW
_L = 16


def _sc_body(idx_hbm, betas_hbm, go_hbm, tr_hbm, bp_hbm,
             betas_out, bp_out, go_out, tr_out,
             idx_v, betas_v, bp_v, go_v, tr_v, sem, sem2, sem3, sem4):
    wid = lax.axis_index("s") * _NC + lax.axis_index("c")
    base = wid * _BPW

    pltpu.sync_copy(idx_hbm.at[pl.ds(base, _BPW)], idx_v)

    def fetch(c, _):
        vec = idx_v[pl.ds(c * _L, _L)]
        for j in range(_L):
            i = c * _L + j
            r = vec[j]
            pltpu.async_copy(bp_hbm.at[pl.ds(r, 1)], bp_v.at[pl.ds(i, 1)],
                             sem)
            pltpu.async_copy(go_hbm.at[pl.ds(r, 1)], go_v.at[pl.ds(i, 1)],
                             sem2)
            pltpu.async_copy(tr_hbm.at[pl.ds(r, 1)], tr_v.at[pl.ds(i, 1)],
                             sem3)
            pltpu.async_copy(betas_hbm.at[pl.ds(0, 1)],
                             betas_v.at[pl.ds(i, 1)], sem4)
        return ()

    lax.fori_loop(0, _BPW // _L, fetch, ())

    pltpu.make_async_copy(bp_hbm.at[pl.ds(0, _BPW)], bp_v, sem).wait()
    pltpu.make_async_copy(go_hbm.at[pl.ds(0, _BPW)], go_v, sem2).wait()
    pltpu.make_async_copy(tr_hbm.at[pl.ds(0, _BPW)], tr_v, sem3).wait()
    pltpu.make_async_copy(betas_hbm.at[pl.ds(0, _BPW)], betas_v, sem4).wait()

    pltpu.sync_copy(bp_v, bp_out.at[pl.ds(base, _BPW)])
    pltpu.sync_copy(go_v, go_out.at[pl.ds(base, _BPW)])
    pltpu.sync_copy(tr_v, tr_out.at[pl.ds(base, _BPW)])
    pltpu.sync_copy(betas_v, betas_out.at[pl.ds(base, _BPW)])


def kernel(idx, betas_w, global_orient_w, transl_w, body_pose_w):
    idx32 = idx.astype(jnp.int32)
    f = pl.kernel(
        _sc_body,
        out_type=(
            jax.ShapeDtypeStruct((_B, 10), jnp.float32),
            jax.ShapeDtypeStruct((_B, 69), jnp.float32),
            jax.ShapeDtypeStruct((_B, 3), jnp.float32),
            jax.ShapeDtypeStruct((_B, 3), jnp.float32),
        ),
        mesh=plsc.VectorSubcoreMesh(core_axis_name="c", subcore_axis_name="s"),
        scratch_types=[
            pltpu.VMEM((_BPW,), jnp.int32),
            pltpu.VMEM((_BPW, 10), jnp.float32),
            pltpu.VMEM((_BPW, 69), jnp.float32),
            pltpu.VMEM((_BPW, 3), jnp.float32),
            pltpu.VMEM((_BPW, 3), jnp.float32),
            pltpu.SemaphoreType.DMA,
            pltpu.SemaphoreType.DMA,
            pltpu.SemaphoreType.DMA,
            pltpu.SemaphoreType.DMA,
        ],
    )
    return f(idx32, betas_w, global_orient_w, transl_w, body_pose_w)


# speedup vs baseline: 1.0023x; 1.0023x over previous
import jax
import jax.numpy as jnp
from jax import lax
from jax.experimental import pallas as pl
from jax.experimental.pallas import tpu as pltpu
from jax.experimental.pallas import tpu_sc as plsc

_NC = 2
_NS = 16
_NW = _NC * _NS
_B = 4096
_BPW = _B // _NW
_L = 16


def _sc_body(idx_hbm, betas_hbm, go_hbm, tr_hbm, bp_hbm,
             betas_out, bp_out, go_out, tr_out,
             idx_v, betas_v, bp_v, go_v, tr_v, sem):
    wid = lax.axis_index("s") * _NC + lax.axis_index("c")
    base = wid * _BPW

    pltpu.sync_copy(idx_hbm.at[pl.ds(base, _BPW)], idx_v)

    def fetch(c, _):
        vec = idx_v[pl.ds(c * _L, _L)]
        for j in range(_L):
            i = c * _L + j
            r = vec[j]
            pltpu.async_copy(bp_hbm.at[pl.ds(r, 1)], bp_v.at[pl.ds(i, 1)],
                             sem)
            pltpu.async_copy(go_hbm.at[pl.ds(r, 1)], go_v.at[pl.ds(i, 1)],
                             sem)
            pltpu.async_copy(tr_hbm.at[pl.ds(r, 1)], tr_v.at[pl.ds(i, 1)],
                             sem)
            pltpu.async_copy(betas_hbm.at[pl.ds(0, 1)],
                             betas_v.at[pl.ds(i, 1)], sem)
        return ()

    lax.fori_loop(0, _BPW // _L, fetch, ())

    pltpu.make_async_copy(bp_hbm.at[pl.ds(0, _BPW)], bp_v, sem).wait()
    pltpu.make_async_copy(go_hbm.at[pl.ds(0, _BPW)], go_v, sem).wait()
    pltpu.make_async_copy(tr_hbm.at[pl.ds(0, _BPW)], tr_v, sem).wait()
    pltpu.make_async_copy(betas_hbm.at[pl.ds(0, _BPW)], betas_v, sem).wait()

    pltpu.sync_copy(bp_v, bp_out.at[pl.ds(base, _BPW)])
    pltpu.sync_copy(go_v, go_out.at[pl.ds(base, _BPW)])
    pltpu.sync_copy(tr_v, tr_out.at[pl.ds(base, _BPW)])
    pltpu.sync_copy(betas_v, betas_out.at[pl.ds(base, _BPW)])


def kernel(idx, betas_w, global_orient_w, transl_w, body_pose_w):
    idx32 = idx.astype(jnp.int32)
    f = pl.kernel(
        _sc_body,
        out_type=(
            jax.ShapeDtypeStruct((_B, 10), jnp.float32),
            jax.ShapeDtypeStruct((_B, 69), jnp.float32),
            jax.ShapeDtypeStruct((_B, 3), jnp.float32),
            jax.ShapeDtypeStruct((_B, 3), jnp.float32),
        ),
        mesh=plsc.VectorSubcoreMesh(core_axis_name="c", subcore_axis_name="s"),
        scratch_types=[
            pltpu.VMEM((_BPW,), jnp.int32),
            pltpu.VMEM((_BPW, 10), jnp.float32),
            pltpu.VMEM((_BPW, 69), jnp.float32),
            pltpu.VMEM((_BPW, 3), jnp.float32),
            pltpu.VMEM((_BPW, 3), jnp.float32),
            pltpu.SemaphoreType.DMA,
        ],
    )
    return f(idx32, betas_w, global_orient_w, transl_w, body_pose_w)
